# baseline (speedup 1.0000x reference)
import jax
import jax.numpy as jnp
from jax import lax
from jax.experimental import pallas as pl
from jax.experimental.pallas import tpu as pltpu
from jax.experimental.pallas import tpu_sc as plsc

BATCH = 4096
CTX = 200
VOCAB = 100000
DIM = 64
NC = 2
NS = 16
NW = NC * NS
CH = 2048
NH = BATCH // CH
L = 16
GRP = CH // L
MASK_HI = -65536


def _emb_body(x_hbm, p_hbm, embp_hbm, posp_hbm, out_hbm,
              etab, ptab, xb, pb, ob, isem, wsem):
    w = lax.axis_index("s") * NC + lax.axis_index("c")
    d0 = 2 * w

    pltpu.sync_copy(embp_hbm.at[pl.ds(w * VOCAB, VOCAB)], etab)
    pltpu.sync_copy(posp_hbm.at[pl.ds(w * CTX, CTX)], ptab)

    def issue_idx(t, q):
        s = pl.ds(q * CH, CH)
        pltpu.async_copy(x_hbm.at[t, s], xb.at[q], isem.at[q])
        pltpu.async_copy(p_hbm.at[t, s], pb.at[q], isem.at[q])

    def wait_idx(q):
        pltpu.make_async_copy(x_hbm.at[0, pl.ds(0, CH)], xb.at[q],
                              isem.at[q]).wait()
        pltpu.make_async_copy(p_hbm.at[0, pl.ds(0, CH)], pb.at[q],
                              isem.at[q]).wait()

    def wait_writes(q):
        for h in range(2):
            pltpu.make_async_copy(ob.at[q, h], out_hbm.at[0, 0, pl.ds(0, CH)],
                                  wsem.at[q]).wait()

    issue_idx(0, 0)

    def outer(up, _):
        for q in range(NH):
            wait_idx(q)
            if q == 0:
                issue_idx(up, 1)
            else:
                @pl.when(up < CTX - 1)
                def _():
                    issue_idx(up + 1, 0)
            @pl.when(up > 0)
            def _():
                wait_writes(q)

            def grp_body(g, _):
                s = pl.ds(g * L, L)
                xi = xb[q, s]
                pi = pb[q, s]
                ew = plsc.load_gather(etab, [xi])
                pw = plsc.load_gather(ptab, [pi])
                e_lo = plsc.bitcast(lax.shift_left(ew, 16), jnp.float32)
                p_lo = plsc.bitcast(lax.shift_left(pw, 16), jnp.float32)
                m = jnp.int32(MASK_HI)
                e_hi = plsc.bitcast(lax.bitwise_and(ew, m), jnp.float32)
                p_hi = plsc.bitcast(lax.bitwise_and(pw, m), jnp.float32)
                ob[q, 0, s] = e_lo + p_lo
                ob[q, 1, s] = e_hi + p_hi
                return 0

            lax.fori_loop(0, GRP, grp_body, 0, unroll=4)

            for h in range(2):
                pltpu.async_copy(ob.at[q, h],
                                 out_hbm.at[up, d0 + h, pl.ds(q * CH, CH)],
                                 wsem.at[q])
        return 0

    lax.fori_loop(0, CTX, outer, 0)

    for q in range(NH):
        wait_writes(q)


def _pack_pairs(table16):
    v = table16.shape[0]
    packed = lax.bitcast_convert_type(
        table16.reshape(v, DIM // 2, 2), jnp.int32)
    return jnp.swapaxes(packed, 0, 1).reshape(-1)


@jax.jit
def kernel(x, _pos, emb_table, pos_table):
    xT = jnp.swapaxes(x, 0, 1)
    pT = jnp.swapaxes(_pos, 0, 1)
    embp = _pack_pairs(emb_table.astype(jnp.bfloat16))
    posp = _pack_pairs(pos_table.astype(jnp.bfloat16))
    k = pl.kernel(
        _emb_body,
        out_type=jax.ShapeDtypeStruct((CTX, DIM, BATCH), jnp.float32),
        mesh=plsc.VectorSubcoreMesh(core_axis_name="c", subcore_axis_name="s"),
        compiler_params=pltpu.CompilerParams(use_tc_tiling_on_sc=False,
                                             needs_layout_passes=False),
        scratch_types=[
            pltpu.VMEM((VOCAB,), jnp.int32),
            pltpu.VMEM((CTX,), jnp.int32),
            pltpu.VMEM((NH, CH), jnp.int32),
            pltpu.VMEM((NH, CH), jnp.int32),
            pltpu.VMEM((NH, 2, CH), jnp.float32),
            pltpu.SemaphoreType.DMA((NH,)),
            pltpu.SemaphoreType.DMA((NH,)),
        ],
    )
    out = k(xT, pT, embp, posp)
    return jnp.transpose(out, (2, 0, 1))


# speedup vs baseline: 1.3405x; 1.3405x over previous
import jax
import jax.numpy as jnp
from jax import lax
from jax.experimental import pallas as pl
from jax.experimental.pallas import tpu as pltpu
from jax.experimental.pallas import tpu_sc as plsc

BATCH = 4096
CTX = 200
VOCAB = 100000
DIM = 64
NC = 2
NS = 16
NW = NC * NS
CH = 2048
NH = BATCH // CH
L = 16
GRP = CH // L
MASK_HI = -65536


def _emb_body(x_hbm, p_hbm, embp_hbm, posp_hbm, out_hbm,
              etab, ptab, xb, pb, ob, isem, wsem):
    w = lax.axis_index("s") * NC + lax.axis_index("c")
    d0 = 2 * w

    pltpu.sync_copy(embp_hbm.at[pl.ds(w * VOCAB, VOCAB)], etab)
    pltpu.sync_copy(posp_hbm.at[pl.ds(w * CTX, CTX)], ptab)

    def issue_idx(t, q):
        s = pl.ds(q * CH, CH)
        pltpu.async_copy(x_hbm.at[t, s], xb.at[q], isem.at[q])
        pltpu.async_copy(p_hbm.at[t, s], pb.at[q], isem.at[q])

    def wait_idx(q):
        pltpu.make_async_copy(x_hbm.at[0, pl.ds(0, CH)], xb.at[q],
                              isem.at[q]).wait()
        pltpu.make_async_copy(p_hbm.at[0, pl.ds(0, CH)], pb.at[q],
                              isem.at[q]).wait()

    def wait_writes(q):
        for h in range(2):
            pltpu.make_async_copy(ob.at[q, h],
                                  out_hbm.at[0, 0, pl.ds(0, CH // 128), 0, :],
                                  wsem.at[q]).wait()

    issue_idx(0, 0)

    def outer(up, _):
        for q in range(NH):
            wait_idx(q)
            if q == 0:
                issue_idx(up, 1)
            else:
                @pl.when(up < CTX - 1)
                def _():
                    issue_idx(up + 1, 0)
            @pl.when(up > 0)
            def _():
                wait_writes(q)

            def grp_body(g, _):
                s = pl.ds(g * L, L)
                xi = xb[q, s]
                pi = pb[q, s]
                ew = plsc.load_gather(etab, [xi])
                pw = plsc.load_gather(ptab, [pi])
                e_lo = plsc.bitcast(lax.shift_left(ew, 16), jnp.float32)
                p_lo = plsc.bitcast(lax.shift_left(pw, 16), jnp.float32)
                m = jnp.int32(MASK_HI)
                e_hi = plsc.bitcast(lax.bitwise_and(ew, m), jnp.float32)
                p_hi = plsc.bitcast(lax.bitwise_and(pw, m), jnp.float32)
                r = lax.shift_right_logical(g, 3)
                c = pl.ds(lax.mul(lax.bitwise_and(g, 7), L), L)
                ob[q, 0, r, c] = e_lo + p_lo
                ob[q, 1, r, c] = e_hi + p_hi
                return 0

            lax.fori_loop(0, GRP, grp_body, 0)

            wg = w // 4
            for h in range(2):
                dd = 2 * (w % 4) + h
                pltpu.async_copy(
                    ob.at[q, h],
                    out_hbm.at[up, wg, pl.ds(q * (CH // 128), CH // 128), dd, :],
                    wsem.at[q])
        return 0

    lax.fori_loop(0, CTX, outer, 0)

    for q in range(NH):
        wait_writes(q)


def _pack_pairs(table16):
    u = lax.bitcast_convert_type(table16, jnp.uint16).astype(jnp.uint32)
    packed = u[:, 0::2] | (u[:, 1::2] << 16)
    return lax.bitcast_convert_type(
        jnp.swapaxes(packed, 0, 1), jnp.int32).reshape(-1)


@jax.jit
def kernel(x, _pos, emb_table, pos_table):
    xT = jnp.swapaxes(x, 0, 1)
    pT = jnp.swapaxes(_pos, 0, 1)
    embp = _pack_pairs(emb_table.astype(jnp.bfloat16))
    posp = _pack_pairs(pos_table.astype(jnp.bfloat16))
    k = pl.kernel(
        _emb_body,
        out_type=jax.ShapeDtypeStruct((CTX, DIM // 8, BATCH // 128, 8, 128),
                                      jnp.float32),
        mesh=plsc.VectorSubcoreMesh(core_axis_name="c", subcore_axis_name="s"),
        compiler_params=pltpu.CompilerParams(use_tc_tiling_on_sc=False,
                                             needs_layout_passes=False),
        scratch_types=[
            pltpu.VMEM((VOCAB,), jnp.int32),
            pltpu.VMEM((CTX,), jnp.int32),
            pltpu.VMEM((NH, CH), jnp.int32),
            pltpu.VMEM((NH, CH), jnp.int32),
            pltpu.VMEM((NH, 2, CH // 128, 128), jnp.float32),
            pltpu.SemaphoreType.DMA((NH,)),
            pltpu.SemaphoreType.DMA((NH,)),
        ],
    )
    out = k(xT, pT, embp, posp)
    return jnp.transpose(out, (2, 4, 0, 1, 3)).reshape(BATCH, CTX, DIM)
